# baseline (speedup 1.0000x reference)
import functools

import jax
import jax.numpy as jnp
from jax import lax
from jax.experimental import pallas as pl
from jax.experimental.pallas import tpu as pltpu
from jax.experimental.pallas import tpu_sc as plsc

N = 10000
E = 320000
D = 128
H = 64
WAUG = 80
NT = N + 16
PAD_NODE = N
NPAD = 10240
HALF = NPAD // 2
NSC = 16
CHUNK = 80
SPT = E // NSC
PIECE = 2000
NPIECES = SPT // PIECE
FCAP = SPT + 8 * CHUNK
MAXCH = FCAP // CHUNK
RPT = HALF // NSC
NBUF = 7
NLOOK = 4


def _proj_body(x_ref, w1_ref, o_ref):
    xb = x_ref[...]
    w = w1_ref[...]
    p = lax.dot_general(xb, w, (((1,), (1,)), ((), ())),
                        preferred_element_type=jnp.float32)
    o_ref[0:N, 0:64] = p
    col = lax.broadcasted_iota(jnp.int32, (N, 16), 1)
    o_ref[0:N, 64:80] = jnp.where(col == 0, 1.0, 0.0)
    o_ref[N:NT, :] = jnp.zeros((NT - N, WAUG), jnp.float32)


def _project(x, W1):
    return pl.pallas_call(
        _proj_body,
        out_shape=jax.ShapeDtypeStruct((NT, WAUG), jnp.float32),
    )(x, W1)


_sc_mesh = plsc.VectorSubcoreMesh(core_axis_name="c", subcore_axis_name="s")


@functools.partial(
    pl.kernel,
    out_type=jax.ShapeDtypeStruct((N,), jnp.float32),
    mesh=_sc_mesh,
    scratch_types=[
        pltpu.VMEM((2, PIECE), jnp.int32),
        pltpu.VMEM((2, PIECE), jnp.int32),
        pltpu.VMEM((FCAP,), jnp.int32),
        pltpu.VMEM((FCAP,), jnp.int32),
        [pltpu.VMEM((CHUNK, WAUG), jnp.float32) for _ in range(NBUF)],
        pltpu.VMEM((CHUNK, WAUG), jnp.float32),
        pltpu.VMEM((H,), jnp.float32),
        pltpu.VMEM((WAUG,), jnp.float32),
        pltpu.VMEM((RPT,), jnp.float32),
        pltpu.VMEM_SHARED((HALF, WAUG), jnp.float32),
        [pltpu.SemaphoreType.DMA for _ in range(NBUF)],
        [pltpu.SemaphoreType.DMA for _ in range(NBUF)],
        [pltpu.SemaphoreType.DMA for _ in range(2)],
        pltpu.SemaphoreType.DMA,
    ],
    compiler_params=pltpu.CompilerParams(use_tc_tiling_on_sc=False,
                                         needs_layout_passes=False),
)
def _sc_fused(xp_hbm, edge_hbm, b1_hbm, pw_hbm, out_hbm,
              rawn, rawh, fn, fh, bufs, zbuf, b1v, pwv, obuf, accum,
              gsems, ssems, psems, zsem):
    cid = lax.axis_index("c")
    sid = lax.axis_index("s")
    lo = cid * HALF
    e0 = sid * SPT
    row0 = sid * RPT

    pltpu.async_copy(b1_hbm, b1v, zsem)
    pltpu.async_copy(pw_hbm, pwv, zsem)
    for p in range(2):
        pltpu.async_copy(edge_hbm.at[0, pl.ds(e0 + p * PIECE, PIECE)],
                         rawn.at[p], psems[p])
        pltpu.async_copy(edge_hbm.at[1, pl.ds(e0 + p * PIECE, PIECE)],
                         rawh.at[p], psems[p])

    zeros16 = jnp.zeros((16,), jnp.float32)

    def _zrow(r, _):
        for c in range(WAUG // 16):
            zbuf[r, pl.ds(c * 16, 16)] = zeros16
        return 0

    lax.fori_loop(0, CHUNK, _zrow, 0)
    for r in range(RPT // CHUNK):
        pltpu.async_copy(zbuf, accum.at[pl.ds(row0 + r * CHUNK, CHUNK)], zsem)

    def _scan_piece(bank, ptr):
        def _it(k, ptr):
            nv = rawn[bank, pl.ds(k * 16, 16)]
            hv = rawh[bank, pl.ds(k * 16, 16)]
            m = (hv >= lo) & (hv < lo + HALF)
            plsc.store_compressed(fn.at[pl.ds(ptr, 16)], nv, mask=m)
            plsc.store_compressed(fh.at[pl.ds(ptr, 16)], hv - lo, mask=m)
            return ptr + plsc.all_reduce_population_count(m)[0]

        return lax.fori_loop(0, PIECE // 16, _it, ptr)

    def _two_pieces(q, ptr):
        for bank in range(2):
            p = 2 * q + bank
            pltpu.make_async_copy(edge_hbm.at[0, pl.ds(e0, PIECE)],
                                  rawn.at[bank], psems[bank]).wait()
            pltpu.make_async_copy(edge_hbm.at[1, pl.ds(e0, PIECE)],
                                  rawh.at[bank], psems[bank]).wait()
            ptr = _scan_piece(bank, ptr)

            @pl.when(p + 2 < NPIECES)
            def _():
                nxt = p + 2
                pltpu.async_copy(
                    edge_hbm.at[0, pl.ds(e0 + nxt * PIECE, PIECE)],
                    rawn.at[bank], psems[bank])
                pltpu.async_copy(
                    edge_hbm.at[1, pl.ds(e0 + nxt * PIECE, PIECE)],
                    rawh.at[bank], psems[bank])
        return ptr

    cnt = lax.fori_loop(0, NPIECES // 2, _two_pieces, jnp.int32(0))

    lane = lax.iota(jnp.int32, 16)
    for k in range(8 * CHUNK // 16):
        fn[pl.ds(cnt + k * 16, 16)] = jnp.full((16,), PAD_NODE, jnp.int32)
        fh[pl.ds(cnt + k * 16, 16)] = (k * 16 + lane) % HALF
    nchunks = (cnt + CHUNK - 1) // CHUNK + NBUF

    def _hslice(j):
        return fh.at[pl.ds(j * CHUNK, CHUNK)]

    def _nslice(j):
        return fn.at[pl.ds(j * CHUNK, CHUNK)]

    for b in range(NLOOK):
        pltpu.async_copy(xp_hbm.at[_nslice(b)], bufs[b], gsems[b])

    pltpu.make_async_copy(b1_hbm, b1v, zsem).wait()
    pltpu.make_async_copy(pw_hbm, pwv, zsem).wait()
    for r in range(RPT // CHUNK):
        pltpu.make_async_copy(zbuf, accum.at[pl.ds(row0 + r * CHUNK, CHUNK)],
                              zsem).wait()
    plsc.subcore_barrier()

    def _body(i, _):
        for b in range(NBUF):
            j = NBUF * i + b
            bp = (b + NLOOK) % NBUF

            @pl.when(j + NLOOK < nchunks)
            def _():
                @pl.when(j >= NBUF - NLOOK)
                def _():
                    pltpu.make_async_copy(
                        bufs[bp], accum.at[_hslice(j - (NBUF - NLOOK))],
                        ssems[bp]).wait()
                pltpu.async_copy(xp_hbm.at[_nslice(j + NLOOK)], bufs[bp],
                                 gsems[bp])

            @pl.when(j < nchunks)
            def _():
                pltpu.make_async_copy(xp_hbm.at[_nslice(j)], bufs[b],
                                      gsems[b]).wait()
                pltpu.async_copy(bufs[b], accum.at[_hslice(j)], ssems[b],
                                 add=True)
        return 0

    lax.fori_loop(0, (MAXCH + NBUF - 1) // NBUF, _body, 0)

    for b in range(NBUF):
        pltpu.make_async_copy(bufs[b], accum.at[_hslice(0)], ssems[b]).wait()
    plsc.subcore_barrier()

    cnt_col = jnp.full((16,), H, jnp.int32)
    b1a = [b1v[pl.ds(k * 16, 16)] for k in range(H // 16)]
    pwa = [pwv[pl.ds(k * 16, 16)] for k in range(WAUG // 16)]
    last_tile = (cid == 1) & (sid == NSC - 1)
    npieces_out = jnp.where(last_tile, 1, RPT // CHUNK)

    def _piece_out(q, _):
        pltpu.sync_copy(accum.at[pl.ds(row0 + q * CHUNK, CHUNK)], zbuf)
        for g in range(CHUNK // 16):
            rows = g * 16 + lax.iota(jnp.int32, 16)
            cgt = plsc.load_gather(zbuf, [rows, cnt_col])
            rinv = 1.0 / jnp.maximum(cgt, 1.0)
            alpha = jnp.zeros((16,), jnp.float32)
            for c in range(H):
                colv = jnp.full((16,), c, jnp.int32)
                f = plsc.load_gather(zbuf, [rows, colv])
                h = jnp.maximum(f * rinv + b1a[c // 16][c % 16], 0.0)
                alpha = alpha + h * pwa[c // 16][c % 16]
            alpha = jnp.clip(alpha + pwa[H // 16][0], -5.0, 5.0)
            sig = 1.0 / (1.0 + jnp.exp(-alpha))
            obuf[pl.ds(q * CHUNK + g * 16, 16)] = sig * 0.9 + 0.1
        return 0

    lax.fori_loop(0, npieces_out, _piece_out, 0)

    g0 = cid * HALF + row0

    @pl.when(jnp.logical_not(last_tile))
    def _():
        pltpu.sync_copy(obuf, out_hbm.at[pl.ds(g0, RPT)])

    @pl.when(last_tile)
    def _():
        pltpu.sync_copy(obuf.at[pl.ds(0, CHUNK)],
                        out_hbm.at[pl.ds(N - CHUNK, CHUNK)])


def kernel(x, edge_index, W1, b1, W2, b2):
    xp_aug = _project(x, W1)
    pw = jnp.concatenate(
        [W2[0], b2, jnp.zeros((WAUG - H - 1,), jnp.float32)])
    return _sc_fused(xp_aug, edge_index, b1, pw)


# speedup vs baseline: 5.3232x; 5.3232x over previous
import functools

import jax
import jax.numpy as jnp
from jax import lax
from jax.experimental import pallas as pl
from jax.experimental.pallas import tpu as pltpu
from jax.experimental.pallas import tpu_sc as plsc

N = 10000
E = 320000
D = 128
H = 64
WAUG = 80
NPAD = 10240
NTILES = 32
CHUNK = 80
EPT = E // NTILES
CHUNKS = EPT // CHUNK
TAIL = EPT - CHUNKS * CHUNK
ROWS_PER_TILE = NPAD // 16


def _proj_body(x_ref, w1_ref, o_ref):
    xb = x_ref[...]
    w = w1_ref[...]
    p = lax.dot_general(xb, w, (((1,), (1,)), ((), ())),
                        preferred_element_type=jnp.float32)
    o_ref[:, 0:64] = p
    col = lax.broadcasted_iota(jnp.int32, (xb.shape[0], 16), 1)
    o_ref[:, 64:80] = jnp.where(col == 0, 1.0, 0.0)


def _project(x, W1):
    return pl.pallas_call(
        _proj_body,
        out_shape=jax.ShapeDtypeStruct((N, WAUG), jnp.float32),
    )(x, W1)


_sc_mesh = plsc.VectorSubcoreMesh(core_axis_name="c", subcore_axis_name="s")


@functools.partial(
    pl.kernel,
    out_type=jax.ShapeDtypeStruct((2, NPAD, WAUG), jnp.float32),
    mesh=_sc_mesh,
    scratch_types=[
        pltpu.VMEM((EPT,), jnp.int32),
        pltpu.VMEM((EPT,), jnp.int32),
        [pltpu.VMEM((CHUNK, WAUG), jnp.float32) for _ in range(7)],
        pltpu.VMEM((CHUNK, WAUG), jnp.float32),
        pltpu.VMEM_SHARED((NPAD, WAUG), jnp.float32),
        [pltpu.SemaphoreType.DMA for _ in range(7)],
        [pltpu.SemaphoreType.DMA for _ in range(7)],
        pltpu.SemaphoreType.DMA,
        pltpu.SemaphoreType.DMA,
    ],
    compiler_params=pltpu.CompilerParams(use_tc_tiling_on_sc=False),
)
def _sc_scatter(xp_hbm, edge_hbm, out_hbm, nidx, hidx, bufs, zbuf, accum,
                gsems, ssems, isem, zsem):
    cid = lax.axis_index("c")
    sid = lax.axis_index("s")
    wid = sid * 2 + cid
    row0 = sid * ROWS_PER_TILE

    e0 = wid * EPT
    pltpu.async_copy(edge_hbm.at[0, pl.ds(e0, EPT)], nidx, isem)
    pltpu.async_copy(edge_hbm.at[1, pl.ds(e0, EPT)], hidx, isem)

    zeros16 = jnp.zeros((16,), jnp.float32)

    def _zrow(r, _):
        for c in range(WAUG // 16):
            zbuf[r, pl.ds(c * 16, 16)] = zeros16
        return 0

    ZREM = ROWS_PER_TILE % CHUNK
    lax.fori_loop(0, CHUNK, _zrow, 0)
    for r in range(ROWS_PER_TILE // CHUNK):
        pltpu.async_copy(zbuf, accum.at[pl.ds(row0 + r * CHUNK, CHUNK)], zsem)
    if ZREM:
        pltpu.async_copy(
            zbuf.at[pl.ds(0, ZREM)],
            accum.at[pl.ds(row0 + (ROWS_PER_TILE // CHUNK) * CHUNK, ZREM)],
            zsem)

    pltpu.make_async_copy(edge_hbm.at[0, pl.ds(e0, EPT)], nidx, isem).wait()
    pltpu.make_async_copy(edge_hbm.at[1, pl.ds(e0, EPT)], hidx, isem).wait()

    NBUF = 7
    NLOOK = 4
    for b in range(NLOOK):
        pltpu.async_copy(xp_hbm.at[nidx.at[pl.ds(b * CHUNK, CHUNK)]],
                         bufs[b], gsems[b])

    for r in range(ROWS_PER_TILE // CHUNK):
        pltpu.make_async_copy(zbuf, accum.at[pl.ds(row0 + r * CHUNK, CHUNK)],
                              zsem).wait()
    if ZREM:
        pltpu.make_async_copy(
            zbuf.at[pl.ds(0, ZREM)],
            accum.at[pl.ds(row0 + (ROWS_PER_TILE // CHUNK) * CHUNK, ZREM)],
            zsem).wait()
    plsc.subcore_barrier()

    def _hslice(j):
        return hidx.at[pl.ds(j * CHUNK, CHUNK)]

    def _nslice(j):
        return nidx.at[pl.ds(j * CHUNK, CHUNK)]

    def _body(i, _):
        for b in range(NBUF):
            j = NBUF * i + b
            bp = (b + NLOOK) % NBUF

            @pl.when(j + NLOOK < CHUNKS)
            def _():
                @pl.when(j >= NBUF - NLOOK)
                def _():
                    pltpu.make_async_copy(
                        bufs[bp], accum.at[_hslice(j - (NBUF - NLOOK))],
                        ssems[bp]).wait()
                pltpu.async_copy(xp_hbm.at[_nslice(j + NLOOK)], bufs[bp],
                                 gsems[bp])

            @pl.when(j < CHUNKS)
            def _():
                pltpu.make_async_copy(xp_hbm.at[_nslice(j)], bufs[b],
                                      gsems[b]).wait()
                pltpu.async_copy(bufs[b], accum.at[_hslice(j)], ssems[b],
                                 add=True)
        return 0

    lax.fori_loop(0, (CHUNKS + NBUF - 1) // NBUF, _body, 0)

    for b in range(NBUF):
        j = CHUNKS - NBUF + b
        pltpu.make_async_copy(bufs[j % NBUF], accum.at[_hslice(j)],
                              ssems[j % NBUF]).wait()

    if TAIL:
        pltpu.sync_copy(xp_hbm.at[nidx.at[pl.ds(CHUNKS * CHUNK, TAIL)]],
                        zbuf.at[pl.ds(0, TAIL)])
        pltpu.sync_copy(zbuf.at[pl.ds(0, TAIL)],
                        accum.at[hidx.at[pl.ds(CHUNKS * CHUNK, TAIL)]],
                        add=True)
    plsc.subcore_barrier()

    pltpu.sync_copy(accum.at[pl.ds(row0, ROWS_PER_TILE)],
                    out_hbm.at[cid, pl.ds(row0, ROWS_PER_TILE)])


RPT2 = NPAD // NTILES
GROUPS_FULL = RPT2 // 16
TAIL_GROUPS = (N - (NTILES - 1) * RPT2) // 16


@functools.partial(
    pl.kernel,
    out_type=jax.ShapeDtypeStruct((N,), jnp.float32),
    mesh=_sc_mesh,
    scratch_types=[
        pltpu.VMEM((RPT2, WAUG), jnp.float32),
        pltpu.VMEM((RPT2, WAUG), jnp.float32),
        pltpu.VMEM((H,), jnp.float32),
        pltpu.VMEM((WAUG,), jnp.float32),
        pltpu.VMEM((RPT2,), jnp.float32),
        pltpu.SemaphoreType.DMA,
    ],
    compiler_params=pltpu.CompilerParams(use_tc_tiling_on_sc=False,
                                         needs_layout_passes=False),
)
def _sc_post(acc_hbm, b1_hbm, pw_hbm, out_hbm, a0, a1, b1v, pwv, obuf, dsem):
    cid = lax.axis_index("c")
    sid = lax.axis_index("s")
    wid = sid * 2 + cid
    r0 = wid * RPT2

    pltpu.async_copy(acc_hbm.at[0, pl.ds(r0, RPT2)], a0, dsem)
    pltpu.async_copy(acc_hbm.at[1, pl.ds(r0, RPT2)], a1, dsem)
    pltpu.async_copy(b1_hbm, b1v, dsem)
    pltpu.async_copy(pw_hbm, pwv, dsem)
    pltpu.make_async_copy(acc_hbm.at[0, pl.ds(r0, RPT2)], a0, dsem).wait()
    pltpu.make_async_copy(acc_hbm.at[1, pl.ds(r0, RPT2)], a1, dsem).wait()
    pltpu.make_async_copy(b1_hbm, b1v, dsem).wait()
    pltpu.make_async_copy(pw_hbm, pwv, dsem).wait()

    cnt_col = jnp.full((16,), H, jnp.int32)
    b1a = [b1v[pl.ds(k * 16, 16)] for k in range(H // 16)]
    pwa = [pwv[pl.ds(k * 16, 16)] for k in range(WAUG // 16)]

    def _one_group(g):
        rows = g * 16 + lax.iota(jnp.int32, 16)
        cnt = (plsc.load_gather(a0, [rows, cnt_col])
               + plsc.load_gather(a1, [rows, cnt_col]))
        rinv = 1.0 / jnp.maximum(cnt, 1.0)
        alpha = jnp.zeros((16,), jnp.float32)
        for c in range(H):
            colv = jnp.full((16,), c, jnp.int32)
            f = (plsc.load_gather(a0, [rows, colv])
                 + plsc.load_gather(a1, [rows, colv]))
            h = jnp.maximum(f * rinv + b1a[c // 16][c % 16], 0.0)
            alpha = alpha + h * pwa[c // 16][c % 16]
        alpha = jnp.clip(alpha + pwa[H // 16][0], -5.0, 5.0)
        sig = 1.0 / (1.0 + jnp.exp(-alpha))
        obuf[pl.ds(g * 16, 16)] = sig * 0.9 + 0.1

    ngroups = jnp.where(wid == NTILES - 1, TAIL_GROUPS, GROUPS_FULL)

    def _group(g, _):
        _one_group(g)
        return 0

    lax.fori_loop(0, ngroups, _group, 0)

    @pl.when(wid < NTILES - 1)
    def _():
        pltpu.sync_copy(obuf, out_hbm.at[pl.ds(r0, RPT2)])

    @pl.when(wid == NTILES - 1)
    def _():
        pltpu.sync_copy(obuf.at[pl.ds(0, TAIL_GROUPS * 16)],
                        out_hbm.at[pl.ds((NTILES - 1) * RPT2,
                                         TAIL_GROUPS * 16)])


def kernel(x, edge_index, W1, b1, W2, b2):
    xp_aug = _project(x, W1)

    acc = _sc_scatter(xp_aug, edge_index)

    pw = jnp.concatenate(
        [W2[0], b2, jnp.zeros((WAUG - H - 1,), jnp.float32)])
    return _sc_post(acc, b1, pw)
